# Initial kernel scaffold; baseline (speedup 1.0000x reference)
import jax
import jax.numpy as jnp
from jax.experimental import pallas as pl


def kernel(history_stack, edge_index, W_ih, W_hh, b_ih, b_hh, gat_W, attn_l, attn_r, gat_bias, dec_W, dec_b):
    raise NotImplementedError("write your pallas kernel here")


# speedup vs baseline: 6.6912x; 6.6912x over previous
import jax
import jax.numpy as jnp
from jax.experimental import pallas as pl
from jax.experimental.pallas import tpu as pltpu


def _adj_kernel(src_ref, dst_ref, c_ref):
    f = c_ref.shape[0]
    e = src_ref.shape[1]
    rows = jax.lax.broadcasted_iota(jnp.int32, (f, e), 0)
    src_oh = (src_ref[...] == rows).astype(jnp.float32)
    dst_oh = (dst_ref[...] == rows).astype(jnp.float32)
    c = jax.lax.dot_general(
        dst_oh, src_oh, (((1,), (1,)), ((), ())),
        preferred_element_type=jnp.float32,
        precision=jax.lax.Precision.HIGHEST)
    eye = (jax.lax.broadcasted_iota(jnp.int32, (f, f), 0) ==
           jax.lax.broadcasted_iota(jnp.int32, (f, f), 1)).astype(jnp.float32)
    c_ref[...] = c + eye


def _gnn_kernel(x_ref, c_ref, wih_ref, bih_ref, whh_ref, bhh_ref,
                gatw_ref, al_ref, ar_ref, gb_ref, decw_ref, decb_ref,
                ans_ref, hid_ref, h_scr):
    t = pl.program_id(1)
    nh = gatw_ref.shape[0]

    @pl.when(t == 0)
    def _init():
        h_scr[...] = jnp.zeros_like(h_scr)

    h = h_scr[...]
    x = x_ref[...][0][:, None, :]

    gi = x * wih_ref[...][None, :, :] + bih_ref[...][None, :, :]
    whh = whh_ref[...]
    gh = bhh_ref[...][None, :, :]
    for k in range(nh):
        gh = gh + h[:, k:k + 1, :] * whh[:, k, :][None, :, :]

    i_r = gi[:, 0:nh, :]
    i_z = gi[:, nh:2 * nh, :]
    i_n = gi[:, 2 * nh:3 * nh, :]
    h_r = gh[:, 0:nh, :]
    h_z = gh[:, nh:2 * nh, :]
    h_n = gh[:, 2 * nh:3 * nh, :]
    r = jax.nn.sigmoid(i_r + h_r)
    z = jax.nn.sigmoid(i_z + h_z)
    n = jnp.tanh(i_n + r * h_n)
    hg = (1.0 - z) * n + z * h

    ansv = jnp.sum(hg * decw_ref[...][None, :, :], axis=1) + decb_ref[...]
    ans_ref[...] = ansv[None, :, :]

    gatw = gatw_ref[...]
    feat = hg[:, 0:1, :] * gatw[0:1, :][:, :, None]
    for k in range(1, nh):
        feat = feat + hg[:, k:k + 1, :] * gatw[k:k + 1, :][:, :, None]

    el = jnp.sum(feat * al_ref[...][None, :, :], axis=1)
    er = jnp.sum(feat * ar_ref[...][None, :, :], axis=1)

    e3 = el[:, None, :] + er[:, :, None]
    e3 = jnp.where(e3 >= 0.0, e3, 0.2 * e3)
    c = c_ref[...]
    mask = (c > 0.0)[None, :, :]
    e3m = jnp.where(mask, e3, -1e30)
    m = jnp.max(e3m, axis=2)
    ex = jnp.exp(e3m - m[:, :, None]) * c[None, :, :]
    ssum = jnp.sum(ex, axis=2)
    alpha = ex * (1.0 / ssum)[:, :, None]

    out = jax.lax.dot_general(
        feat, alpha, (((2,), (2,)), ((0,), (0,))),
        preferred_element_type=jnp.float32,
        precision=jax.lax.Precision.HIGHEST)
    hnew = out + gb_ref[...][None, :, :]
    h_scr[...] = hnew
    hid_ref[:, 0, :, :] = hnew


def kernel(history_stack, edge_index, W_ih, W_hh, b_ih, b_hh, gat_W,
           attn_l, attn_r, gat_bias, dec_W, dec_b):
    B, T, F = history_stack.shape
    H = gat_W.shape[0]
    E = edge_index.shape[1]

    ei = edge_index.astype(jnp.int32)
    src = ei[0].reshape(1, E)
    dst = ei[1].reshape(1, E)

    c = pl.pallas_call(
        _adj_kernel,
        out_shape=jax.ShapeDtypeStruct((F, F), jnp.float32),
    )(src, dst)

    wih_t = jnp.transpose(W_ih[:, :, 0])
    bih_t = jnp.transpose(b_ih)
    whh_t = jnp.transpose(W_hh, (1, 2, 0))
    bhh_t = jnp.transpose(b_hh)
    decw_t = jnp.transpose(dec_W)
    al2 = attn_l.reshape(H, 1)
    ar2 = attn_r.reshape(H, 1)
    gb2 = gat_bias.reshape(H, 1)
    db2 = dec_b.reshape(1, F)

    BB = 32 if B % 32 == 0 else B
    grid = (B // BB, T)
    hs_t = jnp.transpose(history_stack, (1, 0, 2))

    const = lambda b, t: (0, 0)
    const3 = lambda b, t: (0, 0, 0)
    ans_t, hid = pl.pallas_call(
        _gnn_kernel,
        grid=grid,
        in_specs=[
            pl.BlockSpec((1, BB, F), lambda b, t: (t, b, 0)),
            pl.BlockSpec((F, F), const),
            pl.BlockSpec((3 * H, F), const),
            pl.BlockSpec((3 * H, F), const),
            pl.BlockSpec((3 * H, H, F), const3),
            pl.BlockSpec((3 * H, F), const),
            pl.BlockSpec((H, H), const),
            pl.BlockSpec((H, 1), const),
            pl.BlockSpec((H, 1), const),
            pl.BlockSpec((H, 1), const),
            pl.BlockSpec((H, F), const),
            pl.BlockSpec((1, F), const),
        ],
        out_specs=[
            pl.BlockSpec((1, BB, F), lambda b, t: (t, b, 0)),
            pl.BlockSpec((BB, 1, H, F), lambda b, t: (b, t, 0, 0)),
        ],
        out_shape=[
            jax.ShapeDtypeStruct((T, B, F), jnp.float32),
            jax.ShapeDtypeStruct((B, T, H, F), jnp.float32),
        ],
        scratch_shapes=[pltpu.VMEM((BB, H, F), jnp.float32)],
    )(hs_t, c, wih_t, bih_t, whh_t, bhh_t, gat_W,
      al2, ar2, gb2, decw_t, db2)

    ans = jnp.transpose(ans_t, (1, 0, 2))
    hiddens = jnp.transpose(hid, (0, 1, 3, 2))
    return ans, hiddens


# speedup vs baseline: 6.8579x; 1.0249x over previous
import jax
import jax.numpy as jnp
from jax.experimental import pallas as pl
from jax.experimental.pallas import tpu as pltpu


def _adj_kernel(src_ref, dst_ref, c_ref):
    f = c_ref.shape[0]
    e = src_ref.shape[1]
    rows = jax.lax.broadcasted_iota(jnp.int32, (f, e), 0)
    src_oh = (src_ref[...] == rows).astype(jnp.float32)
    dst_oh = (dst_ref[...] == rows).astype(jnp.float32)
    c = jax.lax.dot_general(
        dst_oh, src_oh, (((1,), (1,)), ((), ())),
        preferred_element_type=jnp.float32,
        precision=jax.lax.Precision.HIGHEST)
    eye = (jax.lax.broadcasted_iota(jnp.int32, (f, f), 0) ==
           jax.lax.broadcasted_iota(jnp.int32, (f, f), 1)).astype(jnp.float32)
    cnt = c + eye
    c_ref[...] = jnp.where(cnt > 0.0, jnp.log(cnt), -1e30)


def _gnn_kernel(x_ref, c_ref, wih_ref, bih_ref, whh_ref, bhh_ref,
                gatw_ref, al_ref, ar_ref, gb_ref, decw_ref, decb_ref,
                ans_ref, hid_ref, h_scr):
    t = pl.program_id(1)
    nh = gatw_ref.shape[0]

    @pl.when(t == 0)
    def _init():
        h_scr[...] = jnp.zeros_like(h_scr)

    h = h_scr[...]
    x = x_ref[...][0][:, None, :]

    gi = x * wih_ref[...][None, :, :] + bih_ref[...][None, :, :]
    whh = whh_ref[...]
    gh = bhh_ref[...][None, :, :]
    for k in range(nh):
        gh = gh + h[:, k:k + 1, :] * whh[:, k, :][None, :, :]

    i_r = gi[:, 0:nh, :]
    i_z = gi[:, nh:2 * nh, :]
    i_n = gi[:, 2 * nh:3 * nh, :]
    h_r = gh[:, 0:nh, :]
    h_z = gh[:, nh:2 * nh, :]
    h_n = gh[:, 2 * nh:3 * nh, :]
    r = jax.nn.sigmoid(i_r + h_r)
    z = jax.nn.sigmoid(i_z + h_z)
    n = jnp.tanh(i_n + r * h_n)
    hg = (1.0 - z) * n + z * h

    ansv = jnp.sum(hg * decw_ref[...][None, :, :], axis=1) + decb_ref[...]
    ans_ref[...] = ansv[None, :, :]

    featT = jax.lax.dot_general(
        gatw_ref[...], hg, (((0,), (1,)), ((), ())),
        preferred_element_type=jnp.float32,
        precision=jax.lax.Precision.HIGHEST)

    el = jnp.sum(hg * al_ref[...][None, :, :], axis=1)
    er = jnp.sum(hg * ar_ref[...][None, :, :], axis=1)

    e3 = el[:, None, :] + er[:, :, None]
    e3 = jnp.where(e3 >= 0.0, e3, 0.2 * e3) + c_ref[...][None, :, :]
    m = jnp.max(e3, axis=2)
    ex = jnp.exp(e3 - m[:, :, None])
    ssum = jnp.sum(ex, axis=2)

    out = jax.lax.dot_general(
        featT, ex, (((2,), (2,)), ((1,), (0,))),
        preferred_element_type=jnp.float32,
        precision=jax.lax.Precision.HIGHEST)
    hnew = out * (1.0 / ssum)[:, None, :] + gb_ref[...][None, :, :]
    h_scr[...] = hnew
    hid_ref[:, 0, :, :] = hnew


def kernel(history_stack, edge_index, W_ih, W_hh, b_ih, b_hh, gat_W,
           attn_l, attn_r, gat_bias, dec_W, dec_b):
    B, T, F = history_stack.shape
    H = gat_W.shape[0]
    E = edge_index.shape[1]

    ei = edge_index.astype(jnp.int32)
    src = ei[0].reshape(1, E)
    dst = ei[1].reshape(1, E)

    c = pl.pallas_call(
        _adj_kernel,
        out_shape=jax.ShapeDtypeStruct((F, F), jnp.float32),
    )(src, dst)

    wih_t = jnp.transpose(W_ih[:, :, 0])
    bih_t = jnp.transpose(b_ih)
    whh_t = jnp.transpose(W_hh, (1, 2, 0))
    bhh_t = jnp.transpose(b_hh)
    decw_t = jnp.transpose(dec_W)
    al2 = (gat_W @ attn_l).reshape(H, 1)
    ar2 = (gat_W @ attn_r).reshape(H, 1)
    gb2 = gat_bias.reshape(H, 1)
    db2 = dec_b.reshape(1, F)

    BB = 32 if B % 32 == 0 else B
    grid = (B // BB, T)
    hs_t = jnp.transpose(history_stack, (1, 0, 2))

    const = lambda b, t: (0, 0)
    const3 = lambda b, t: (0, 0, 0)
    ans_t, hid = pl.pallas_call(
        _gnn_kernel,
        grid=grid,
        in_specs=[
            pl.BlockSpec((1, BB, F), lambda b, t: (t, b, 0)),
            pl.BlockSpec((F, F), const),
            pl.BlockSpec((3 * H, F), const),
            pl.BlockSpec((3 * H, F), const),
            pl.BlockSpec((3 * H, H, F), const3),
            pl.BlockSpec((3 * H, F), const),
            pl.BlockSpec((H, H), const),
            pl.BlockSpec((H, 1), const),
            pl.BlockSpec((H, 1), const),
            pl.BlockSpec((H, 1), const),
            pl.BlockSpec((H, F), const),
            pl.BlockSpec((1, F), const),
        ],
        out_specs=[
            pl.BlockSpec((1, BB, F), lambda b, t: (t, b, 0)),
            pl.BlockSpec((BB, 1, H, F), lambda b, t: (b, t, 0, 0)),
        ],
        out_shape=[
            jax.ShapeDtypeStruct((T, B, F), jnp.float32),
            jax.ShapeDtypeStruct((B, T, H, F), jnp.float32),
        ],
        scratch_shapes=[pltpu.VMEM((BB, H, F), jnp.float32)],
        compiler_params=pltpu.CompilerParams(
            dimension_semantics=("parallel", "arbitrary"),
        ),
    )(hs_t, c, wih_t, bih_t, whh_t, bhh_t, gat_W,
      al2, ar2, gb2, decw_t, db2)

    ans = jnp.transpose(ans_t, (1, 0, 2))
    hiddens = jnp.transpose(hid, (0, 1, 3, 2))
    return ans, hiddens


# speedup vs baseline: 8.6367x; 1.2594x over previous
import jax
import jax.numpy as jnp
from jax.experimental import pallas as pl
from jax.experimental.pallas import tpu as pltpu


def _adj_kernel(src_ref, dst_ref, c_ref):
    f = c_ref.shape[0]
    e = src_ref.shape[1]
    rows = jax.lax.broadcasted_iota(jnp.int32, (f, e), 0)
    src_oh = (src_ref[...] == rows).astype(jnp.float32)
    dst_oh = (dst_ref[...] == rows).astype(jnp.float32)
    c = jax.lax.dot_general(
        dst_oh, src_oh, (((1,), (1,)), ((), ())),
        preferred_element_type=jnp.float32,
        precision=jax.lax.Precision.HIGHEST)
    eye = (jax.lax.broadcasted_iota(jnp.int32, (f, f), 0) ==
           jax.lax.broadcasted_iota(jnp.int32, (f, f), 1)).astype(jnp.float32)
    cnt = c + eye
    c_ref[...] = jnp.where(cnt > 0.0, jnp.log(cnt), -1e30)


def _gnn_kernel(x_ref, c_ref, wih_ref, bih_ref, whh_ref, bhh_ref,
                gatw_ref, al_ref, ar_ref, gb_ref, decw_ref, decb_ref,
                ans_ref, hid_ref, h_scr):
    t = pl.program_id(1)
    nh = gatw_ref.shape[0]

    @pl.when(t == 0)
    def _init():
        h_scr[...] = jnp.zeros_like(h_scr)

    h = h_scr[...]
    x = x_ref[...][0][:, None, :]

    gi = x * wih_ref[...][None, :, :] + bih_ref[...][None, :, :]
    whh = whh_ref[...]
    gh = bhh_ref[...][None, :, :]
    for k in range(nh):
        gh = gh + h[:, k:k + 1, :] * whh[:, k, :][None, :, :]

    i_r = gi[:, 0:nh, :]
    i_z = gi[:, nh:2 * nh, :]
    i_n = gi[:, 2 * nh:3 * nh, :]
    h_r = gh[:, 0:nh, :]
    h_z = gh[:, nh:2 * nh, :]
    h_n = gh[:, 2 * nh:3 * nh, :]
    r = jax.nn.sigmoid(i_r + h_r)
    z = jax.nn.sigmoid(i_z + h_z)
    n = jnp.tanh(i_n + r * h_n)
    hg = (1.0 - z) * n + z * h

    ansv = jnp.sum(hg * decw_ref[...][None, :, :], axis=1) + decb_ref[...]
    ans_ref[...] = ansv[None, :, :]

    featT = jax.lax.dot_general(
        gatw_ref[...], hg, (((0,), (1,)), ((), ())),
        preferred_element_type=jnp.float32,
        precision=jax.lax.Precision.HIGHEST)

    el = jnp.sum(hg * al_ref[...][None, :, :], axis=1)
    er = jnp.sum(hg * ar_ref[...][None, :, :], axis=1)

    e3 = el[:, None, :] + er[:, :, None]
    e3 = jnp.where(e3 >= 0.0, e3, 0.2 * e3) + c_ref[...][None, :, :]
    m = jnp.max(e3, axis=2)
    ex = jnp.exp(e3 - m[:, :, None])
    ssum = jnp.sum(ex, axis=2)

    out = jax.lax.dot_general(
        featT, ex, (((2,), (2,)), ((1,), (0,))),
        preferred_element_type=jnp.float32,
        precision=jax.lax.Precision.DEFAULT)
    hnew = out * (1.0 / ssum)[:, None, :] + gb_ref[...][None, :, :]
    h_scr[...] = hnew
    hid_ref[:, 0, :, :] = hnew


def kernel(history_stack, edge_index, W_ih, W_hh, b_ih, b_hh, gat_W,
           attn_l, attn_r, gat_bias, dec_W, dec_b):
    B, T, F = history_stack.shape
    H = gat_W.shape[0]
    E = edge_index.shape[1]

    ei = edge_index.astype(jnp.int32)
    src = ei[0].reshape(1, E)
    dst = ei[1].reshape(1, E)

    c = pl.pallas_call(
        _adj_kernel,
        out_shape=jax.ShapeDtypeStruct((F, F), jnp.float32),
    )(src, dst)

    wih_t = jnp.transpose(W_ih[:, :, 0])
    bih_t = jnp.transpose(b_ih)
    whh_t = jnp.transpose(W_hh, (1, 2, 0))
    bhh_t = jnp.transpose(b_hh)
    decw_t = jnp.transpose(dec_W)
    al2 = (gat_W @ attn_l).reshape(H, 1)
    ar2 = (gat_W @ attn_r).reshape(H, 1)
    gb2 = gat_bias.reshape(H, 1)
    db2 = dec_b.reshape(1, F)

    BB = 32 if B % 32 == 0 else B
    grid = (B // BB, T)
    hs_t = jnp.transpose(history_stack, (1, 0, 2))

    const = lambda b, t: (0, 0)
    const3 = lambda b, t: (0, 0, 0)
    ans_t, hid = pl.pallas_call(
        _gnn_kernel,
        grid=grid,
        in_specs=[
            pl.BlockSpec((1, BB, F), lambda b, t: (t, b, 0)),
            pl.BlockSpec((F, F), const),
            pl.BlockSpec((3 * H, F), const),
            pl.BlockSpec((3 * H, F), const),
            pl.BlockSpec((3 * H, H, F), const3),
            pl.BlockSpec((3 * H, F), const),
            pl.BlockSpec((H, H), const),
            pl.BlockSpec((H, 1), const),
            pl.BlockSpec((H, 1), const),
            pl.BlockSpec((H, 1), const),
            pl.BlockSpec((H, F), const),
            pl.BlockSpec((1, F), const),
        ],
        out_specs=[
            pl.BlockSpec((1, BB, F), lambda b, t: (t, b, 0)),
            pl.BlockSpec((BB, 1, H, F), lambda b, t: (b, t, 0, 0)),
        ],
        out_shape=[
            jax.ShapeDtypeStruct((T, B, F), jnp.float32),
            jax.ShapeDtypeStruct((B, T, H, F), jnp.float32),
        ],
        scratch_shapes=[pltpu.VMEM((BB, H, F), jnp.float32)],
        compiler_params=pltpu.CompilerParams(
            dimension_semantics=("parallel", "arbitrary"),
        ),
    )(hs_t, c, wih_t, bih_t, whh_t, bhh_t, gat_W,
      al2, ar2, gb2, decw_t, db2)

    ans = jnp.transpose(ans_t, (1, 0, 2))
    hiddens = jnp.transpose(hid, (0, 1, 3, 2))
    return ans, hiddens


# speedup vs baseline: 10.1759x; 1.1782x over previous
import jax
import jax.numpy as jnp
from jax.experimental import pallas as pl
from jax.experimental.pallas import tpu as pltpu


def _adj_kernel(src_ref, dst_ref, c_ref):
    f = c_ref.shape[0]
    e = src_ref.shape[1]
    rows = jax.lax.broadcasted_iota(jnp.int32, (f, e), 0)
    src_oh = (src_ref[...] == rows).astype(jnp.float32)
    dst_oh = (dst_ref[...] == rows).astype(jnp.float32)
    c = jax.lax.dot_general(
        dst_oh, src_oh, (((1,), (1,)), ((), ())),
        preferred_element_type=jnp.float32,
        precision=jax.lax.Precision.HIGHEST)
    eye = (jax.lax.broadcasted_iota(jnp.int32, (f, f), 0) ==
           jax.lax.broadcasted_iota(jnp.int32, (f, f), 1)).astype(jnp.float32)
    cnt = c + eye
    c_ref[...] = jnp.where(cnt > 0.0, jnp.log(cnt), -1e30)


def _gnn_kernel(x_ref, c_ref, wih_ref, bihn_ref, whh_ref, bsum_ref,
                gatw_ref, al_ref, ar_ref, gb_ref, decw_ref, decb_ref,
                ans_ref, hid_ref, h_scr):
    t = pl.program_id(1)
    nh = gatw_ref.shape[0]

    @pl.when(t == 0)
    def _init():
        h_scr[...] = jnp.zeros_like(h_scr)

    h = h_scr[...]
    x = x_ref[...][0][:, None, :]

    a = bsum_ref[...][None, :, :] + x * wih_ref[...][None, :, :]
    for k in range(nh):
        a = a + h[:, k:k + 1, :] * whh_ref[k][None, :, :]

    r = jax.nn.sigmoid(a[:, 0:nh, :])
    z = jax.nn.sigmoid(a[:, nh:2 * nh, :])
    i_n = x * wih_ref[2 * nh:3 * nh, :][None, :, :] + bihn_ref[...][None, :, :]
    h_n = a[:, 2 * nh:3 * nh, :] - i_n
    n = jnp.tanh(i_n + r * h_n)
    hg = (1.0 - z) * n + z * h

    ansv = jnp.sum(hg * decw_ref[...][None, :, :], axis=1) + decb_ref[...]
    ans_ref[...] = ansv[None, :, :]

    featT = jax.lax.dot_general(
        gatw_ref[...], hg, (((0,), (1,)), ((), ())),
        preferred_element_type=jnp.float32,
        precision=jax.lax.Precision.DEFAULT)

    el = jnp.sum(hg * al_ref[...][None, :, :], axis=1)
    er = jnp.sum(hg * ar_ref[...][None, :, :], axis=1)

    e3 = el[:, None, :] + er[:, :, None]
    e3 = jnp.where(e3 >= 0.0, e3, 0.2 * e3) + c_ref[...][None, :, :]
    m = jnp.max(e3, axis=2)
    ex = jnp.exp(e3 - m[:, :, None])
    ssum = jnp.sum(ex, axis=2)

    out = jax.lax.dot_general(
        featT, ex, (((2,), (2,)), ((1,), (0,))),
        preferred_element_type=jnp.float32,
        precision=jax.lax.Precision.DEFAULT)
    hnew = out * (1.0 / ssum)[:, None, :] + gb_ref[...][None, :, :]
    h_scr[...] = hnew
    hid_ref[:, 0, :, :] = hnew


def kernel(history_stack, edge_index, W_ih, W_hh, b_ih, b_hh, gat_W,
           attn_l, attn_r, gat_bias, dec_W, dec_b):
    B, T, F = history_stack.shape
    H = gat_W.shape[0]
    E = edge_index.shape[1]

    ei = edge_index.astype(jnp.int32)
    src = ei[0].reshape(1, E)
    dst = ei[1].reshape(1, E)

    c = pl.pallas_call(
        _adj_kernel,
        out_shape=jax.ShapeDtypeStruct((F, F), jnp.float32),
    )(src, dst)

    wih_t = jnp.transpose(W_ih[:, :, 0])
    bihn_t = jnp.transpose(b_ih[:, 2 * H:3 * H])
    whh_t = jnp.transpose(W_hh, (2, 1, 0))
    bsum_t = jnp.transpose(b_ih + b_hh)
    decw_t = jnp.transpose(dec_W)
    al2 = (gat_W @ attn_l).reshape(H, 1)
    ar2 = (gat_W @ attn_r).reshape(H, 1)
    gb2 = gat_bias.reshape(H, 1)
    db2 = dec_b.reshape(1, F)

    BB = 32 if B % 32 == 0 else B
    grid = (B // BB, T)
    hs_t = jnp.transpose(history_stack, (1, 0, 2))

    const = lambda b, t: (0, 0)
    const3 = lambda b, t: (0, 0, 0)
    ans_t, hid = pl.pallas_call(
        _gnn_kernel,
        grid=grid,
        in_specs=[
            pl.BlockSpec((1, BB, F), lambda b, t: (t, b, 0)),
            pl.BlockSpec((F, F), const),
            pl.BlockSpec((3 * H, F), const),
            pl.BlockSpec((H, F), const),
            pl.BlockSpec((H, 3 * H, F), const3),
            pl.BlockSpec((3 * H, F), const),
            pl.BlockSpec((H, H), const),
            pl.BlockSpec((H, 1), const),
            pl.BlockSpec((H, 1), const),
            pl.BlockSpec((H, 1), const),
            pl.BlockSpec((H, F), const),
            pl.BlockSpec((1, F), const),
        ],
        out_specs=[
            pl.BlockSpec((1, BB, F), lambda b, t: (t, b, 0)),
            pl.BlockSpec((BB, 1, H, F), lambda b, t: (b, t, 0, 0)),
        ],
        out_shape=[
            jax.ShapeDtypeStruct((T, B, F), jnp.float32),
            jax.ShapeDtypeStruct((B, T, H, F), jnp.float32),
        ],
        scratch_shapes=[pltpu.VMEM((BB, H, F), jnp.float32)],
        compiler_params=pltpu.CompilerParams(
            dimension_semantics=("parallel", "arbitrary"),
        ),
    )(hs_t, c, wih_t, bihn_t, whh_t, bsum_t, gat_W,
      al2, ar2, gb2, decw_t, db2)

    ans = jnp.transpose(ans_t, (1, 0, 2))
    hiddens = jnp.transpose(hid, (0, 1, 3, 2))
    return ans, hiddens
